# Initial kernel scaffold; baseline (speedup 1.0000x reference)
import jax
import jax.numpy as jnp
from jax.experimental import pallas as pl


def kernel(x, r_ij, neighbors, neighbor_mask, f_ij, W1, b1, W2, b2, Win, Wf, bf, Wd, bd):
    raise NotImplementedError("write your pallas kernel here")


# speedup vs baseline: 6.4610x; 6.4610x over previous
import functools

import jax
import jax.numpy as jnp
from jax import lax
from jax.experimental import pallas as pl
from jax.experimental.pallas import tpu as pltpu
from jax.experimental.pallas import tpu_sc as plsc

_LOG2 = 0.6931471805599453
_CUTOFF = 5.0

_NC = 2
_NS = 16
_NW = _NC * _NS


def _ssp(v):
    return jax.nn.softplus(v) - _LOG2


def _in2f_body(x_ref, w_ref, y_ref):
    y_ref[...] = jnp.dot(x_ref[...], w_ref[...],
                         preferred_element_type=jnp.float32)


def _in2f(x2d, Win):
    M, D = x2d.shape
    F = Win.shape[1]
    T = 1000
    return pl.pallas_call(
        _in2f_body,
        grid=(M // T,),
        in_specs=[
            pl.BlockSpec((T, D), lambda i: (i, 0)),
            pl.BlockSpec((D, F), lambda i: (0, 0)),
        ],
        out_specs=pl.BlockSpec((T, F), lambda i: (i, 0)),
        out_shape=jax.ShapeDtypeStruct((M, F), jnp.float32),
    )(x2d, Win)


def _sc_gather(table, idx3d):
    NW, NCH, CH = idx3d.shape
    F = table.shape[1]
    mesh = plsc.VectorSubcoreMesh(core_axis_name="c", subcore_axis_name="s",
                                  num_cores=_NC, num_subcores=_NS)

    @functools.partial(
        pl.kernel,
        out_type=jax.ShapeDtypeStruct((NW, NCH, CH, F), jnp.float32),
        mesh=mesh,
        scratch_types=[
            pltpu.VMEM((2, CH), jnp.int32),
            pltpu.VMEM((2, CH, F), jnp.float32),
            pltpu.SemaphoreType.DMA,
            pltpu.SemaphoreType.DMA,
        ],
    )
    def k(table_hbm, idx_hbm, out_hbm, idx_v, rows_v, gsem, ssem):
        w = lax.axis_index("s") * _NC + lax.axis_index("c")

        pltpu.sync_copy(idx_hbm.at[w, 0], idx_v.at[0])
        pltpu.async_copy(table_hbm.at[idx_v.at[0]], rows_v.at[0], gsem)

        def body(j, _):
            s = lax.rem(j, 2)
            o = lax.rem(j + 1, 2)

            @pl.when(j + 1 < NCH)
            def _():
                @pl.when(j >= 1)
                def _():
                    pltpu.make_async_copy(rows_v.at[o], out_hbm.at[w, j - 1],
                                          ssem).wait()
                pltpu.sync_copy(idx_hbm.at[w, j + 1], idx_v.at[o])
                pltpu.async_copy(table_hbm.at[idx_v.at[o]], rows_v.at[o],
                                 gsem)

            pltpu.make_async_copy(table_hbm.at[idx_v.at[s]], rows_v.at[s],
                                  gsem).wait()
            pltpu.async_copy(rows_v.at[s], out_hbm.at[w, j], ssem)
            return 0

        lax.fori_loop(0, NCH, body, 0, unroll=False)
        if NCH >= 2:
            pltpu.make_async_copy(rows_v.at[lax.rem(NCH - 2, 2)],
                                  out_hbm.at[w, NCH - 2], ssem).wait()
        pltpu.make_async_copy(rows_v.at[lax.rem(NCH - 1, 2)],
                              out_hbm.at[w, NCH - 1], ssem).wait()

    return k(table, idx3d)


def _fused_body(f_ref, r_ref, m_ref, g_ref, w1_ref, b1_ref, w2_ref, b2_ref,
                wf_ref, bf_ref, wd_ref, bd_ref, o_ref, *, T, NBR, S, F):
    f = f_ref[...].reshape(T * NBR, S)
    h = _ssp(jnp.dot(f, w1_ref[...], preferred_element_type=jnp.float32)
             + b1_ref[...])
    w = jnp.dot(h, w2_ref[...], preferred_element_type=jnp.float32) + b2_ref[...]
    r = r_ref[...]
    c = 0.5 * (jnp.cos(r * (jnp.pi / _CUTOFF)) + 1.0)
    c = c * (r < _CUTOFF).astype(jnp.float32) * m_ref[...]
    w = w.reshape(T, NBR, F) * c[..., None]
    agg = jnp.sum(w * g_ref[...], axis=1)
    a = _ssp(jnp.dot(agg, wf_ref[...], preferred_element_type=jnp.float32)
             + bf_ref[...])
    o_ref[...] = jnp.dot(a, wd_ref[...], preferred_element_type=jnp.float32) \
        + bd_ref[...]


def _fused(f_ij, r_ij, mask, g, W1, b1, W2, b2, Wf, bf, Wd, bd):
    M, NBR, S = f_ij.shape
    F = W2.shape[1]
    A = Wd.shape[1]
    T = 200
    body = functools.partial(_fused_body, T=T, NBR=NBR, S=S, F=F)
    full = lambda i: (0, 0)
    return pl.pallas_call(
        body,
        grid=(M // T,),
        in_specs=[
            pl.BlockSpec((T, NBR, S), lambda i: (i, 0, 0)),
            pl.BlockSpec((T, NBR), lambda i: (i, 0)),
            pl.BlockSpec((T, NBR), lambda i: (i, 0)),
            pl.BlockSpec((T, NBR, F), lambda i: (i, 0, 0)),
            pl.BlockSpec((S, F), full),
            pl.BlockSpec((1, F), full),
            pl.BlockSpec((F, F), full),
            pl.BlockSpec((1, F), full),
            pl.BlockSpec((F, A), full),
            pl.BlockSpec((1, A), full),
            pl.BlockSpec((A, A), full),
            pl.BlockSpec((1, A), full),
        ],
        out_specs=pl.BlockSpec((T, A), lambda i: (i, 0)),
        out_shape=jax.ShapeDtypeStruct((M, A), jnp.float32),
    )(f_ij, r_ij, mask, g, W1, b1.reshape(1, -1), W2, b2.reshape(1, -1),
      Wf, bf.reshape(1, -1), Wd, bd.reshape(1, -1))


def kernel(x, r_ij, neighbors, neighbor_mask, f_ij, W1, b1, W2, b2, Win,
           Wf, bf, Wd, bd):
    B, N, NBR = neighbors.shape
    D = x.shape[-1]
    S = f_ij.shape[-1]
    M = B * N
    E = M * NBR

    y = _in2f(x.reshape(M, D), Win)

    idx = (neighbors + (jnp.arange(B, dtype=jnp.int32) * N)[:, None, None])
    per_w = E // _NW
    CH = 80
    idx3d = idx.reshape(_NW, per_w // CH, CH)

    g = _sc_gather(y, idx3d).reshape(M, NBR, -1)

    v = _fused(f_ij.reshape(M, NBR, S), r_ij.reshape(M, NBR),
               neighbor_mask.reshape(M, NBR), g,
               W1, b1, W2, b2, Wf, bf, Wd, bd)
    return v.reshape(B, N, -1)
